# baseline (speedup 1.0000x reference)
import functools

import jax
import jax.numpy as jnp
from jax import lax
from jax.experimental import pallas as pl
from jax.experimental.pallas import tpu as pltpu
from jax.experimental.pallas import tpu_sc as plsc

VOCAB = 1000000
EMB = 64
BATCH = 16384
CTX = 20

NC = 2
NS = 16
NW = NC * NS
BPW = BATCH // NW
CHUNK = 128
NJ = BPW // CHUNK
INV_CTX = 1.0 / CTX

def _make_mesh():
    return plsc.VectorSubcoreMesh(
        core_axis_name="c", subcore_axis_name="s", num_cores=NC, num_subcores=NS
    )


_scratch = [
    pltpu.VMEM((BPW * CTX,), jnp.int32),
    pltpu.VMEM((CTX * NJ, CHUNK), jnp.int32),
    pltpu.VMEM((BPW, EMB), jnp.float32),
    pltpu.SemaphoreType.DMA,
]


def _cbow_body(x_hbm, table_hbm, out_hbm, idx_flat, idx_t, acc, sem):
    wid = lax.axis_index("s") * NC + lax.axis_index("c")
    base = wid * BPW

    pltpu.sync_copy(x_hbm.at[pl.ds(base * CTX, BPW * CTX)], idx_flat)

    iota16 = lax.iota(jnp.int32, 16)
    step = iota16 * CTX

    @pl.loop(0, CTX)
    def _transpose(g):
        for j in range(NJ):
            for t in range(CHUNK // 16):
                lanes = step + ((j * CHUNK + t * 16) * CTX + g)
                vals = plsc.load_gather(idx_flat, [lanes])
                idx_t[g * NJ + j, pl.ds(t * 16, 16)] = vals

    for j in range(NJ):
        pltpu.async_copy(
            table_hbm.at[idx_t.at[j]], acc.at[pl.ds(j * CHUNK, CHUNK)], sem
        )
    for j in range(NJ):
        pltpu.make_async_copy(
            table_hbm.at[idx_t.at[j]], acc.at[pl.ds(j * CHUNK, CHUNK)], sem
        ).wait()

    @pl.loop(1, CTX)
    def _accumulate(g):
        for j in range(NJ):
            pltpu.async_copy(
                table_hbm.at[idx_t.at[g * NJ + j]],
                acc.at[pl.ds(j * CHUNK, CHUNK)],
                sem,
                add=True,
            )
        for j in range(NJ):
            pltpu.make_async_copy(
                table_hbm.at[idx_t.at[g * NJ + j]],
                acc.at[pl.ds(j * CHUNK, CHUNK)],
                sem,
            ).wait()

    @pl.loop(0, BPW)
    def _scale(r):
        for v in range(EMB // 16):
            sl = pl.ds(v * 16, 16)
            acc[r, sl] = acc[r, sl] * INV_CTX

    pltpu.sync_copy(acc, out_hbm.at[pl.ds(base, BPW)])


_cbow_sc_cache = []


def _get_cbow_sc():
    if not _cbow_sc_cache:
        _cbow_sc_cache.append(
            pl.kernel(
                _cbow_body,
                mesh=_make_mesh(),
                out_type=jax.ShapeDtypeStruct((BATCH, EMB), jnp.float32),
                scratch_types=_scratch,
                compiler_params=pltpu.CompilerParams(
                    needs_layout_passes=False, use_tc_tiling_on_sc=False
                ),
            )
        )
    return _cbow_sc_cache[0]


def kernel(x, embedding_table):
    return _get_cbow_sc()(x.reshape(BATCH * CTX), embedding_table)


# speedup vs baseline: 1.0118x; 1.0118x over previous
import functools

import jax
import jax.numpy as jnp
from jax import lax
from jax.experimental import pallas as pl
from jax.experimental.pallas import tpu as pltpu
from jax.experimental.pallas import tpu_sc as plsc

VOCAB = 1000000
EMB = 64
BATCH = 16384
CTX = 20

NC = 2
NS = 16
NW = NC * NS
BPW = BATCH // NW
CHUNK = 128
NJ = BPW // CHUNK
INV_CTX = 1.0 / CTX


def _make_mesh():
    return plsc.VectorSubcoreMesh(
        core_axis_name="c", subcore_axis_name="s", num_cores=NC, num_subcores=NS
    )


_scratch = [
    pltpu.VMEM((BPW * CTX,), jnp.int32),
    pltpu.VMEM((CTX * NJ, CHUNK), jnp.int32),
    pltpu.VMEM((BPW, EMB), jnp.float32),
    pltpu.VMEM((BPW, EMB), jnp.float32),
    pltpu.SemaphoreType.DMA,
    pltpu.SemaphoreType.DMA,
]


def _cbow_body(x_hbm, table_hbm, out_hbm, idx_flat, idx_t, acc0, acc1, sem0, sem1):
    wid = lax.axis_index("s") * NC + lax.axis_index("c")
    base = wid * BPW
    accs = (acc0, acc1)
    sems = (sem0, sem1)

    pltpu.sync_copy(x_hbm.at[pl.ds(base * CTX, BPW * CTX)], idx_flat)

    iota16 = lax.iota(jnp.int32, 16)
    step = iota16 * CTX

    def transpose(g):
        for j in range(NJ):
            for t in range(CHUNK // 16):
                lanes = step + ((j * CHUNK + t * 16) * CTX + g)
                idx_t[g * NJ + j, pl.ds(t * 16, 16)] = plsc.load_gather(
                    idx_flat, [lanes]
                )

    def fire(g, acc, add):
        for j in range(NJ):
            pltpu.async_copy(
                table_hbm.at[idx_t.at[g * NJ + j]],
                acc.at[pl.ds(j * CHUNK, CHUNK)],
                sems[0] if acc is acc0 else sems[1],
                add=add,
            )

    def drain(g, acc):
        for j in range(NJ):
            pltpu.make_async_copy(
                table_hbm.at[idx_t.at[g * NJ + j]],
                acc.at[pl.ds(j * CHUNK, CHUNK)],
                sems[0] if acc is acc0 else sems[1],
            ).wait()

    transpose(0)
    fire(0, acc0, add=False)
    transpose(1)
    fire(1, acc1, add=False)

    @pl.loop(2, CTX)
    def _steady(g):
        transpose(g)
        parity = g % 2

        @pl.when(parity == 0)
        def _():
            drain(g - 2, acc0)
            fire(g, acc0, add=True)

        @pl.when(parity == 1)
        def _():
            drain(g - 2, acc1)
            fire(g, acc1, add=True)

    drain(CTX - 2, acc0)
    drain(CTX - 1, acc1)

    @pl.loop(0, BPW)
    def _scale(r):
        for v in range(EMB // 16):
            sl = pl.ds(v * 16, 16)
            acc0[r, sl] = (acc0[r, sl] + acc1[r, sl]) * INV_CTX

    pltpu.sync_copy(acc0, out_hbm.at[pl.ds(base, BPW)])


_cbow_sc_cache = []


def _get_cbow_sc():
    if not _cbow_sc_cache:
        _cbow_sc_cache.append(
            pl.kernel(
                _cbow_body,
                mesh=_make_mesh(),
                out_type=jax.ShapeDtypeStruct((BATCH, EMB), jnp.float32),
                scratch_types=_scratch,
                compiler_params=pltpu.CompilerParams(
                    needs_layout_passes=False, use_tc_tiling_on_sc=False
                ),
            )
        )
    return _cbow_sc_cache[0]


def kernel(x, embedding_table):
    return _get_cbow_sc()(x.reshape(BATCH * CTX), embedding_table)


# speedup vs baseline: 1.0200x; 1.0081x over previous
import functools

import jax
import jax.numpy as jnp
from jax import lax
from jax.experimental import pallas as pl
from jax.experimental.pallas import tpu as pltpu
from jax.experimental.pallas import tpu_sc as plsc

VOCAB = 1000000
EMB = 64
BATCH = 16384
CTX = 20

NC = 2
NS = 16
NW = NC * NS
BPW = BATCH // NW
CHUNK = 128
NJ = BPW // CHUNK
INV_CTX = 1.0 / CTX


def _make_mesh():
    return plsc.VectorSubcoreMesh(
        core_axis_name="c", subcore_axis_name="s", num_cores=NC, num_subcores=NS
    )


_scratch = [
    pltpu.VMEM((BPW * CTX,), jnp.int32),
    pltpu.VMEM((CTX * NJ, CHUNK), jnp.int32),
    pltpu.VMEM((BPW, EMB), jnp.float32),
    pltpu.SemaphoreType.DMA,
    pltpu.SemaphoreType.DMA,
]


def _cbow_body(x_hbm, table_hbm, out_hbm, idx_flat, idx_t, acc, isem, gsem):
    wid = lax.axis_index("s") * NC + lax.axis_index("c")
    base = wid * BPW

    idx_cp = pltpu.async_copy(
        x_hbm.at[pl.ds(base * CTX, BPW * CTX)], idx_flat, isem
    )

    zeros = jnp.zeros((16,), jnp.float32)

    @pl.loop(0, BPW)
    def _zero(r):
        for v in range(EMB // 16):
            acc[r, pl.ds(v * 16, 16)] = zeros

    idx_cp.wait()

    iota16 = lax.iota(jnp.int32, 16)
    step = iota16 * CTX

    @pl.loop(0, CTX)
    def _launch(g):
        for j in range(NJ):
            for t in range(CHUNK // 16):
                lanes = step + ((j * CHUNK + t * 16) * CTX + g)
                idx_t[g * NJ + j, pl.ds(t * 16, 16)] = plsc.load_gather(
                    idx_flat, [lanes]
                )
        for j in range(NJ):
            pltpu.async_copy(
                table_hbm.at[idx_t.at[g * NJ + j]],
                acc.at[pl.ds(j * CHUNK, CHUNK)],
                gsem,
                add=True,
            )

    @pl.loop(0, CTX * NJ)
    def _drain(i):
        pltpu.make_async_copy(
            table_hbm.at[idx_t.at[0]], acc.at[pl.ds(0, CHUNK)], gsem
        ).wait()

    @pl.loop(0, BPW)
    def _scale(r):
        for v in range(EMB // 16):
            sl = pl.ds(v * 16, 16)
            acc[r, sl] = acc[r, sl] * INV_CTX

    pltpu.sync_copy(acc, out_hbm.at[pl.ds(base, BPW)])


_cbow_sc_cache = []


def _get_cbow_sc():
    if not _cbow_sc_cache:
        _cbow_sc_cache.append(
            pl.kernel(
                _cbow_body,
                mesh=_make_mesh(),
                out_type=jax.ShapeDtypeStruct((BATCH, EMB), jnp.float32),
                scratch_types=_scratch,
                compiler_params=pltpu.CompilerParams(
                    needs_layout_passes=False, use_tc_tiling_on_sc=False
                ),
            )
        )
    return _cbow_sc_cache[0]


def kernel(x, embedding_table):
    return _get_cbow_sc()(x.reshape(BATCH * CTX), embedding_table)
